# baseline (speedup 1.0000x reference)
import jax
import jax.numpy as jnp
from jax import lax
from jax.experimental import pallas as pl
from jax.experimental.pallas import tpu as pltpu
from jax.experimental.pallas import tpu_sc as plsc

_B, _N = 8, 4096
_NB_SC = 2
_NB_TC = _B - _NB_SC
_L = 16
_W = 32
_QPW = (_NB_SC * _N) // _W
_PARTS = _N // _QPW
_QG = 4
_UNROLL = 2
_NBLK = 256


def _tc_body(x1_ref, x2t_ref, d1_ref, i1_ref, d2_ref, i2_ref):
    i = pl.program_id(1)
    x1 = x1_ref[0]
    x2t = x2t_ref[0]
    m = x2t.shape[1]

    ab = jax.lax.dot_general(
        x1, x2t, dimension_numbers=(((1,), (0,)), ((), ())),
        preferred_element_type=jnp.float32)
    aa = jnp.sum(x1 * x1, axis=1, keepdims=True)
    bb = jnp.sum(x2t * x2t, axis=0, keepdims=True)
    d = jnp.maximum(aa + bb - 2.0 * ab, 0.0)

    rmin = jnp.min(d, axis=1, keepdims=True)
    lane = jax.lax.broadcasted_iota(jnp.int32, d.shape, 1)
    ridx = jnp.min(jnp.where(d == rmin, lane, jnp.int32(m)),
                   axis=1, keepdims=True)
    d1_ref[0] = rmin
    i1_ref[0] = ridx

    cmin = jnp.min(d, axis=0, keepdims=True)
    row = jax.lax.broadcasted_iota(jnp.int32, d.shape, 0) + i * _NBLK
    cidx = jnp.min(jnp.where(d == cmin, row, jnp.int32(1 << 30)),
                   axis=0, keepdims=True)

    @pl.when(i == 0)
    def _():
        d2_ref[0] = cmin
        i2_ref[0] = cidx

    @pl.when(i != 0)
    def _():
        prev_d = d2_ref[0]
        prev_i = i2_ref[0]
        take = cmin < prev_d
        d2_ref[0] = jnp.where(take, cmin, prev_d)
        i2_ref[0] = jnp.where(take, cidx, prev_i)


def _tc_chamfer(xyz1, xyz2):
    b, n, _ = xyz1.shape
    m = xyz2.shape[1]
    x2t = jnp.transpose(xyz2, (0, 2, 1))
    d1, i1, d2, i2 = pl.pallas_call(
        _tc_body,
        grid=(b, n // _NBLK),
        in_specs=[
            pl.BlockSpec((1, _NBLK, 3), lambda bi, ti: (bi, ti, 0)),
            pl.BlockSpec((1, 3, m), lambda bi, ti: (bi, 0, 0)),
        ],
        out_specs=[
            pl.BlockSpec((1, _NBLK, 1), lambda bi, ti: (bi, ti, 0)),
            pl.BlockSpec((1, _NBLK, 1), lambda bi, ti: (bi, ti, 0)),
            pl.BlockSpec((1, 1, m), lambda bi, ti: (bi, 0, 0)),
            pl.BlockSpec((1, 1, m), lambda bi, ti: (bi, 0, 0)),
        ],
        out_shape=[
            jax.ShapeDtypeStruct((b, n, 1), jnp.float32),
            jax.ShapeDtypeStruct((b, n, 1), jnp.int32),
            jax.ShapeDtypeStruct((b, 1, m), jnp.float32),
            jax.ShapeDtypeStruct((b, 1, m), jnp.int32),
        ],
    )(xyz1, x2t)
    return d1[:, :, 0], d2[:, 0, :], i1[:, :, 0], i2[:, 0, :]


def _perm(v, idx):
    dnums = lax.GatherDimensionNumbers(
        offset_dims=(), collapsed_slice_dims=(0,), start_index_map=(0,))
    return lax.gather(v, jnp.reshape(idx, (_L, 1)), dnums, slice_sizes=(1,),
                      mode=lax.GatherScatterMode.PROMISE_IN_BOUNDS)


def _bcast(v, lane):
    return _perm(v, lax.iota(jnp.int32, _L) * 0 + lane)


def _lanemin(v):
    lanes = lax.iota(jnp.int32, _L)
    r = v
    for stride in (8, 4, 2, 1):
        r = jnp.minimum(r, _perm(r, lanes ^ stride))
    return r


def _rbf16(x):
    u = lax.bitcast_convert_type(x, jnp.uint32)
    r = u + jnp.uint32(0x7FFF) + ((u >> jnp.uint32(16)) & jnp.uint32(1))
    return lax.bitcast_convert_type(r & jnp.uint32(0xFFFF0000), jnp.float32)


def _prep(cx_ref, cy_ref, cz_ref, rx_ref, ry_ref, rz_ref, bb_ref, n):
    def body(t, carry):
        sl = pl.ds(t * _L, _L)
        cx = cx_ref[sl]
        cy = cy_ref[sl]
        cz = cz_ref[sl]
        rx_ref[sl] = _rbf16(cx)
        ry_ref[sl] = _rbf16(cy)
        rz_ref[sl] = _rbf16(cz)
        bb_ref[sl] = (cx * cx + cy * cy) + cz * cz
        return carry
    lax.fori_loop(0, n // _L, body, 0)


def _scan_dir(qx_ref, qy_ref, qz_ref, qrx_ref, qry_ref, qrz_ref, q0,
              crx_ref, cry_ref, crz_ref, cbb_ref,
              od_ref, oi_ref, nq, nc):
    nch = nc // _L
    lanes = lax.iota(jnp.int32, _L)
    big = jnp.full((_L,), jnp.float32(3.0e38))
    izero = jnp.zeros((_L,), jnp.int32)

    def block16(b, carry):
        qsl = pl.ds(q0 + b * _L, _L)
        qxv = qx_ref[qsl]
        qyv = qy_ref[qsl]
        qzv = qz_ref[qsl]
        q2v16 = (qxv * qxv + qyv * qyv) + qzv * qzv
        qrxv = qrx_ref[qsl]
        qryv = qry_ref[qsl]
        qrzv = qrz_ref[qsl]
        dacc = jnp.zeros((_L,), jnp.float32)
        iacc = izero
        for sub in range(_L // _QG):
            bx = [2.0 * _bcast(qrxv, sub * _QG + k) for k in range(_QG)]
            by = [2.0 * _bcast(qryv, sub * _QG + k) for k in range(_QG)]
            bz = [2.0 * _bcast(qrzv, sub * _QG + k) for k in range(_QG)]
            q2 = [_bcast(q2v16, sub * _QG + k) for k in range(_QG)]
            c0 = [0.0 - q2[k] for k in range(_QG)]

            def chunk(t, c, bx=bx, by=by, bz=bz, c0=c0):
                rms = list(c[:_QG])
                ris = list(c[_QG:])
                for u in range(_UNROLL):
                    tu = t * _UNROLL + u if _UNROLL > 1 else t
                    sl = pl.ds(tu * _L, _L)
                    cx = crx_ref[sl]
                    cy = cry_ref[sl]
                    cz = crz_ref[sl]
                    bb = cbb_ref[sl]
                    tv = jnp.full((_L,), tu, dtype=jnp.int32)
                    for k in range(_QG):
                        ab2 = (bx[k] * cx + by[k] * cy) + bz[k] * cz
                        e = jnp.maximum(bb - ab2, c0[k])
                        better = e < rms[k]
                        rms[k] = jnp.where(better, e, rms[k])
                        ris[k] = jnp.where(better, tv, ris[k])
                return tuple(rms + ris)

            fin = lax.fori_loop(0, nch // _UNROLL, chunk,
                                tuple([big] * _QG + [izero] * _QG))
            for k in range(_QG):
                rm = fin[k]
                ri = fin[_QG + k]
                pos = sub * _QG + k
                mvalv = _lanemin(rm)
                gidx = ri * _L + lanes
                cand = jnp.where(rm == mvalv, gidx,
                                 jnp.full((_L,), jnp.int32(1 << 30)))
                bidxv = _lanemin(cand)
                sel = lanes == pos
                dacc = jnp.where(sel, q2[k] + mvalv, dacc)
                iacc = jnp.where(sel, bidxv, iacc)
        od_ref[pl.ds(b * _L, _L)] = dacc
        oi_ref[pl.ds(b * _L, _L)] = iacc
        return carry

    lax.fori_loop(0, nq // _L, block16, 0)


def _worker_id():
    return lax.axis_index("c") * 16 + lax.axis_index("s")


def _sc_chamfer(x1x, x1y, x1z, x2x, x2y, x2z,
                d1_ref, d2_ref, i1_ref, i2_ref,
                c1x, c1y, c1z, c2x, c2y, c2z,
                r1x, r1y, r1z, r2x, r2y, r2z, b1, b2, od, oi):
    w = _worker_id()
    batch = w // _PARTS
    part = w % _PARTS
    cbase = batch * _N
    pltpu.sync_copy(x1x.at[pl.ds(cbase, _N)], c1x)
    pltpu.sync_copy(x1y.at[pl.ds(cbase, _N)], c1y)
    pltpu.sync_copy(x1z.at[pl.ds(cbase, _N)], c1z)
    pltpu.sync_copy(x2x.at[pl.ds(cbase, _N)], c2x)
    pltpu.sync_copy(x2y.at[pl.ds(cbase, _N)], c2y)
    pltpu.sync_copy(x2z.at[pl.ds(cbase, _N)], c2z)
    _prep(c1x, c1y, c1z, r1x, r1y, r1z, b1, _N)
    _prep(c2x, c2y, c2z, r2x, r2y, r2z, b2, _N)
    q0 = part * _QPW
    obase = w * _QPW

    _scan_dir(c1x, c1y, c1z, r1x, r1y, r1z, q0,
              r2x, r2y, r2z, b2, od, oi, _QPW, _N)
    pltpu.sync_copy(od, d1_ref.at[pl.ds(obase, _QPW)])
    pltpu.sync_copy(oi, i1_ref.at[pl.ds(obase, _QPW)])

    _scan_dir(c2x, c2y, c2z, r2x, r2y, r2z, q0,
              r1x, r1y, r1z, b1, od, oi, _QPW, _N)
    pltpu.sync_copy(od, d2_ref.at[pl.ds(obase, _QPW)])
    pltpu.sync_copy(oi, i2_ref.at[pl.ds(obase, _QPW)])


def _sc_chamfer_call(xyz1, xyz2):
    b, n, _ = xyz1.shape
    x1 = jnp.transpose(xyz1, (2, 0, 1)).reshape(3, b * n)
    x2 = jnp.transpose(xyz2, (2, 0, 1)).reshape(3, b * n)
    mesh = plsc.VectorSubcoreMesh(core_axis_name="c", subcore_axis_name="s",
                                  num_cores=2, num_subcores=16)
    f = pl.kernel(
        _sc_chamfer,
        out_type=[
            jax.ShapeDtypeStruct((b * n,), jnp.float32),
            jax.ShapeDtypeStruct((b * n,), jnp.float32),
            jax.ShapeDtypeStruct((b * n,), jnp.int32),
            jax.ShapeDtypeStruct((b * n,), jnp.int32),
        ],
        mesh=mesh,
        scratch_types=(
            [pltpu.VMEM((n,), jnp.float32) for _ in range(14)]
            + [pltpu.VMEM((_QPW,), jnp.float32),
               pltpu.VMEM((_QPW,), jnp.int32)]),
    )
    d1, d2, i1, i2 = f(x1[0], x1[1], x1[2], x2[0], x2[1], x2[2])
    return (d1.reshape(b, n), d2.reshape(b, n),
            i1.reshape(b, n), i2.reshape(b, n))


def kernel(xyz1, xyz2):
    td1, td2, ti1, ti2 = _tc_chamfer(xyz1[:_NB_TC], xyz2[:_NB_TC])
    sd1, sd2, si1, si2 = _sc_chamfer_call(xyz1[_NB_TC:], xyz2[_NB_TC:])
    return (jnp.concatenate([td1, sd1], axis=0),
            jnp.concatenate([td2, sd2], axis=0),
            jnp.concatenate([ti1, si1], axis=0),
            jnp.concatenate([ti2, si2], axis=0))


# speedup vs baseline: 1.0182x; 1.0182x over previous
import jax
import jax.numpy as jnp
from jax import lax
from jax.experimental import pallas as pl
from jax.experimental.pallas import tpu as pltpu
from jax.experimental.pallas import tpu_sc as plsc

_B, _N = 8, 4096
_NB_SC = 2
_NB_TC = _B - _NB_SC
_L = 16
_W = 32
_QPW = (_NB_SC * _N) // _W
_PARTS = _N // _QPW
_QG = 4
_UNROLL = 1
_NBLK = 256


def _tc_body(x1_ref, x2t_ref, d1_ref, i1_ref, d2_ref, i2_ref):
    i = pl.program_id(1)
    x1 = x1_ref[0]
    x2t = x2t_ref[0]
    m = x2t.shape[1]

    ab = jax.lax.dot_general(
        x1, x2t, dimension_numbers=(((1,), (0,)), ((), ())),
        preferred_element_type=jnp.float32)
    aa = jnp.sum(x1 * x1, axis=1, keepdims=True)
    bb = jnp.sum(x2t * x2t, axis=0, keepdims=True)
    d = jnp.maximum(aa + bb - 2.0 * ab, 0.0)

    rmin = jnp.min(d, axis=1, keepdims=True)
    lane = jax.lax.broadcasted_iota(jnp.int32, d.shape, 1)
    ridx = jnp.min(jnp.where(d == rmin, lane, jnp.int32(m)),
                   axis=1, keepdims=True)
    d1_ref[0] = rmin
    i1_ref[0] = ridx

    cmin = jnp.min(d, axis=0, keepdims=True)
    row = jax.lax.broadcasted_iota(jnp.int32, d.shape, 0) + i * _NBLK
    cidx = jnp.min(jnp.where(d == cmin, row, jnp.int32(1 << 30)),
                   axis=0, keepdims=True)

    @pl.when(i == 0)
    def _():
        d2_ref[0] = cmin
        i2_ref[0] = cidx

    @pl.when(i != 0)
    def _():
        prev_d = d2_ref[0]
        prev_i = i2_ref[0]
        take = cmin < prev_d
        d2_ref[0] = jnp.where(take, cmin, prev_d)
        i2_ref[0] = jnp.where(take, cidx, prev_i)


def _tc_chamfer(xyz1, xyz2):
    b, n, _ = xyz1.shape
    m = xyz2.shape[1]
    x2t = jnp.transpose(xyz2, (0, 2, 1))
    d1, i1, d2, i2 = pl.pallas_call(
        _tc_body,
        grid=(b, n // _NBLK),
        in_specs=[
            pl.BlockSpec((1, _NBLK, 3), lambda bi, ti: (bi, ti, 0)),
            pl.BlockSpec((1, 3, m), lambda bi, ti: (bi, 0, 0)),
        ],
        out_specs=[
            pl.BlockSpec((1, _NBLK, 1), lambda bi, ti: (bi, ti, 0)),
            pl.BlockSpec((1, _NBLK, 1), lambda bi, ti: (bi, ti, 0)),
            pl.BlockSpec((1, 1, m), lambda bi, ti: (bi, 0, 0)),
            pl.BlockSpec((1, 1, m), lambda bi, ti: (bi, 0, 0)),
        ],
        out_shape=[
            jax.ShapeDtypeStruct((b, n, 1), jnp.float32),
            jax.ShapeDtypeStruct((b, n, 1), jnp.int32),
            jax.ShapeDtypeStruct((b, 1, m), jnp.float32),
            jax.ShapeDtypeStruct((b, 1, m), jnp.int32),
        ],
    )(xyz1, x2t)
    return d1[:, :, 0], d2[:, 0, :], i1[:, :, 0], i2[:, 0, :]


def _perm(v, idx):
    dnums = lax.GatherDimensionNumbers(
        offset_dims=(), collapsed_slice_dims=(0,), start_index_map=(0,))
    return lax.gather(v, jnp.reshape(idx, (_L, 1)), dnums, slice_sizes=(1,),
                      mode=lax.GatherScatterMode.PROMISE_IN_BOUNDS)


def _bcast(v, lane):
    return _perm(v, lax.iota(jnp.int32, _L) * 0 + lane)


def _lanemin(v):
    lanes = lax.iota(jnp.int32, _L)
    r = v
    for stride in (8, 4, 2, 1):
        r = jnp.minimum(r, _perm(r, lanes ^ stride))
    return r


def _rbf16(x):
    u = lax.bitcast_convert_type(x, jnp.uint32)
    r = u + jnp.uint32(0x7FFF) + ((u >> jnp.uint32(16)) & jnp.uint32(1))
    return lax.bitcast_convert_type(r & jnp.uint32(0xFFFF0000), jnp.float32)


def _prep(cx_ref, cy_ref, cz_ref, rx_ref, ry_ref, rz_ref, bb_ref, n):
    def body(t, carry):
        sl = pl.ds(t * _L, _L)
        cx = cx_ref[sl]
        cy = cy_ref[sl]
        cz = cz_ref[sl]
        rx_ref[sl] = _rbf16(cx)
        ry_ref[sl] = _rbf16(cy)
        rz_ref[sl] = _rbf16(cz)
        bb_ref[sl] = (cx * cx + cy * cy) + cz * cz
        return carry
    lax.fori_loop(0, n // _L, body, 0)


def _scan_fused(qx_ref, qy_ref, qz_ref, qrx_ref, qry_ref, qrz_ref, q0,
                crx_ref, cry_ref, crz_ref, cbb_ref,
                od_ref, oi_ref, colv_ref, coli_ref, nq, nc):
    nch = nc // _L
    lanes = lax.iota(jnp.int32, _L)
    big = jnp.full((_L,), jnp.float32(3.0e38))
    izero = jnp.zeros((_L,), jnp.int32)

    def block16(b, carry):
        qsl = pl.ds(q0 + b * _L, _L)
        qxv = qx_ref[qsl]
        qyv = qy_ref[qsl]
        qzv = qz_ref[qsl]
        aav16 = (qxv * qxv + qyv * qyv) + qzv * qzv
        qrxv = qrx_ref[qsl]
        qryv = qry_ref[qsl]
        qrzv = qrz_ref[qsl]
        dacc = jnp.zeros((_L,), jnp.float32)
        iacc = izero
        for sub in range(_L // _QG):
            bx = [2.0 * _bcast(qrxv, sub * _QG + k) for k in range(_QG)]
            by = [2.0 * _bcast(qryv, sub * _QG + k) for k in range(_QG)]
            bz = [2.0 * _bcast(qrzv, sub * _QG + k) for k in range(_QG)]
            av = [_bcast(aav16, sub * _QG + k) for k in range(_QG)]
            rid = [q0 + b * _L + (sub * _QG + k) for k in range(_QG)]

            def chunk(t, c, bx=bx, by=by, bz=bz, av=av, rid=rid):
                sl = pl.ds(t * _L, _L)
                cx = crx_ref[sl]
                cy = cry_ref[sl]
                cz = crz_ref[sl]
                bb = cbb_ref[sl]
                cv = colv_ref[sl]
                ci = coli_ref[sl]
                tv = jnp.full((_L,), t, dtype=jnp.int32)
                rms = list(c[:_QG])
                ris = list(c[_QG:])
                for k in range(_QG):
                    ab2 = (bx[k] * cx + by[k] * cy) + bz[k] * cz
                    d = jnp.maximum((av[k] + bb) - ab2, 0.0)
                    rbet = d < rms[k]
                    rms[k] = jnp.where(rbet, d, rms[k])
                    ris[k] = jnp.where(rbet, tv, ris[k])
                    cbet = d < cv
                    cv = jnp.where(cbet, d, cv)
                    ci = jnp.where(cbet,
                                   jnp.full((_L,), rid[k], dtype=jnp.int32),
                                   ci)
                colv_ref[sl] = cv
                coli_ref[sl] = ci
                return tuple(rms + ris)

            fin = lax.fori_loop(0, nch, chunk,
                                tuple([big] * _QG + [izero] * _QG))
            for k in range(_QG):
                rm = fin[k]
                ri = fin[_QG + k]
                pos = sub * _QG + k
                mvalv = _lanemin(rm)
                gidx = ri * _L + lanes
                cand = jnp.where(rm == mvalv, gidx,
                                 jnp.full((_L,), jnp.int32(1 << 30)))
                bidxv = _lanemin(cand)
                sel = lanes == pos
                dacc = jnp.where(sel, mvalv, dacc)
                iacc = jnp.where(sel, bidxv, iacc)
        od_ref[pl.ds(b * _L, _L)] = dacc
        oi_ref[pl.ds(b * _L, _L)] = iacc
        return carry

    lax.fori_loop(0, nq // _L, block16, 0)


def _worker_id():
    return lax.axis_index("c") * 16 + lax.axis_index("s")


def _sc_chamfer(x1x, x1y, x1z, x2x, x2y, x2z,
                d1_ref, d2_ref, i1_ref, i2_ref,
                c1x, c1y, c1z, c2x, c2y, c2z,
                r1x, r1y, r1z, r2x, r2y, r2z, b1, b2,
                colv, coli, od, oi, tv2, ti2, shv, shi):
    w = _worker_id()
    batch = w // _PARTS
    part = w % _PARTS
    cbase = batch * _N
    pltpu.sync_copy(x1x.at[pl.ds(cbase, _N)], c1x)
    pltpu.sync_copy(x1y.at[pl.ds(cbase, _N)], c1y)
    pltpu.sync_copy(x1z.at[pl.ds(cbase, _N)], c1z)
    pltpu.sync_copy(x2x.at[pl.ds(cbase, _N)], c2x)
    pltpu.sync_copy(x2y.at[pl.ds(cbase, _N)], c2y)
    pltpu.sync_copy(x2z.at[pl.ds(cbase, _N)], c2z)
    _prep(c1x, c1y, c1z, r1x, r1y, r1z, b1, _N)
    _prep(c2x, c2y, c2z, r2x, r2y, r2z, b2, _N)

    big = jnp.full((_L,), jnp.float32(3.0e38))
    izero = jnp.zeros((_L,), jnp.int32)

    def initcol(t, carry):
        sl = pl.ds(t * _L, _L)
        colv[sl] = big
        coli[sl] = izero
        return carry
    lax.fori_loop(0, _N // _L, initcol, 0)

    q0 = part * _QPW
    obase = w * _QPW

    _scan_fused(c1x, c1y, c1z, r1x, r1y, r1z, q0,
                r2x, r2y, r2z, b2, od, oi, colv, coli, _QPW, _N)
    pltpu.sync_copy(od, d1_ref.at[pl.ds(obase, _QPW)])
    pltpu.sync_copy(oi, i1_ref.at[pl.ds(obase, _QPW)])

    pltpu.sync_copy(colv, shv.at[part])
    pltpu.sync_copy(coli, shi.at[part])
    plsc.subcore_barrier()

    csl = pl.ds(part * _QPW, _QPW)
    pltpu.sync_copy(shv.at[0, csl], od)
    pltpu.sync_copy(shi.at[0, csl], oi)

    def merge_partial(p, carry):
        pltpu.sync_copy(shv.at[p, csl], tv2)
        pltpu.sync_copy(shi.at[p, csl], ti2)

        def merge_chunk(t, c):
            sl = pl.ds(t * _L, _L)
            cur = od[sl]
            new = tv2[sl]
            take = new < cur
            od[sl] = jnp.where(take, new, cur)
            oi[sl] = jnp.where(take, ti2[sl], oi[sl])
            return c
        lax.fori_loop(0, _QPW // _L, merge_chunk, 0)
        return carry
    lax.fori_loop(1, _PARTS, merge_partial, 0)

    pltpu.sync_copy(od, d2_ref.at[pl.ds(obase, _QPW)])
    pltpu.sync_copy(oi, i2_ref.at[pl.ds(obase, _QPW)])


def _sc_chamfer_call(xyz1, xyz2):
    b, n, _ = xyz1.shape
    x1 = jnp.transpose(xyz1, (2, 0, 1)).reshape(3, b * n)
    x2 = jnp.transpose(xyz2, (2, 0, 1)).reshape(3, b * n)
    mesh = plsc.VectorSubcoreMesh(core_axis_name="c", subcore_axis_name="s",
                                  num_cores=2, num_subcores=16)
    f = pl.kernel(
        _sc_chamfer,
        out_type=[
            jax.ShapeDtypeStruct((b * n,), jnp.float32),
            jax.ShapeDtypeStruct((b * n,), jnp.float32),
            jax.ShapeDtypeStruct((b * n,), jnp.int32),
            jax.ShapeDtypeStruct((b * n,), jnp.int32),
        ],
        mesh=mesh,
        scratch_types=(
            [pltpu.VMEM((n,), jnp.float32) for _ in range(14)]
            + [pltpu.VMEM((n,), jnp.float32),
               pltpu.VMEM((n,), jnp.int32),
               pltpu.VMEM((_QPW,), jnp.float32),
               pltpu.VMEM((_QPW,), jnp.int32),
               pltpu.VMEM((_QPW,), jnp.float32),
               pltpu.VMEM((_QPW,), jnp.int32),
               pltpu.VMEM_SHARED((_PARTS, n), jnp.float32),
               pltpu.VMEM_SHARED((_PARTS, n), jnp.int32)]),
    )
    d1, d2, i1, i2 = f(x1[0], x1[1], x1[2], x2[0], x2[1], x2[2])
    return (d1.reshape(b, n), d2.reshape(b, n),
            i1.reshape(b, n), i2.reshape(b, n))


def kernel(xyz1, xyz2):
    td1, td2, ti1, ti2 = _tc_chamfer(xyz1[:_NB_TC], xyz2[:_NB_TC])
    sd1, sd2, si1, si2 = _sc_chamfer_call(xyz1[_NB_TC:], xyz2[_NB_TC:])
    return (jnp.concatenate([td1, sd1], axis=0),
            jnp.concatenate([td2, sd2], axis=0),
            jnp.concatenate([ti1, si1], axis=0),
            jnp.concatenate([ti2, si2], axis=0))


# speedup vs baseline: 1.0837x; 1.0643x over previous
import jax
import jax.numpy as jnp
from jax import lax
from jax.experimental import pallas as pl
from jax.experimental.pallas import tpu as pltpu
from jax.experimental.pallas import tpu_sc as plsc

_B, _N = 8, 4096
_NB_SC = 2
_NB_TC = _B - _NB_SC
_L = 16
_W = 32
_QPW = (_NB_SC * _N) // _W
_PARTS = _N // _QPW
_QG = 4
_UNROLL = 1
_NBLK = 512


def _tc_body(x1_ref, x2t_ref, d1_ref, i1_ref, d2_ref, i2_ref):
    i = pl.program_id(1)
    x1 = x1_ref[0]
    x2t = x2t_ref[0]
    m = x2t.shape[1]

    ab = jax.lax.dot_general(
        x1, x2t, dimension_numbers=(((1,), (0,)), ((), ())),
        preferred_element_type=jnp.float32)
    aa = jnp.sum(x1 * x1, axis=1, keepdims=True)
    bb = jnp.sum(x2t * x2t, axis=0, keepdims=True)
    d = jnp.maximum(aa + bb - 2.0 * ab, 0.0)

    rmin = jnp.min(d, axis=1, keepdims=True)
    lane = jax.lax.broadcasted_iota(jnp.int32, d.shape, 1)
    ridx = jnp.min(jnp.where(d == rmin, lane, jnp.int32(m)),
                   axis=1, keepdims=True)
    d1_ref[0] = rmin
    i1_ref[0] = ridx

    cmin = jnp.min(d, axis=0, keepdims=True)
    row = jax.lax.broadcasted_iota(jnp.int32, d.shape, 0) + i * _NBLK
    cidx = jnp.min(jnp.where(d == cmin, row, jnp.int32(1 << 30)),
                   axis=0, keepdims=True)

    @pl.when(i == 0)
    def _():
        d2_ref[0] = cmin
        i2_ref[0] = cidx

    @pl.when(i != 0)
    def _():
        prev_d = d2_ref[0]
        prev_i = i2_ref[0]
        take = cmin < prev_d
        d2_ref[0] = jnp.where(take, cmin, prev_d)
        i2_ref[0] = jnp.where(take, cidx, prev_i)


def _tc_chamfer(xyz1, xyz2):
    b, n, _ = xyz1.shape
    m = xyz2.shape[1]
    x2t = jnp.transpose(xyz2, (0, 2, 1))
    d1, i1, d2, i2 = pl.pallas_call(
        _tc_body,
        grid=(b, n // _NBLK),
        in_specs=[
            pl.BlockSpec((1, _NBLK, 3), lambda bi, ti: (bi, ti, 0)),
            pl.BlockSpec((1, 3, m), lambda bi, ti: (bi, 0, 0)),
        ],
        out_specs=[
            pl.BlockSpec((1, _NBLK, 1), lambda bi, ti: (bi, ti, 0)),
            pl.BlockSpec((1, _NBLK, 1), lambda bi, ti: (bi, ti, 0)),
            pl.BlockSpec((1, 1, m), lambda bi, ti: (bi, 0, 0)),
            pl.BlockSpec((1, 1, m), lambda bi, ti: (bi, 0, 0)),
        ],
        out_shape=[
            jax.ShapeDtypeStruct((b, n, 1), jnp.float32),
            jax.ShapeDtypeStruct((b, n, 1), jnp.int32),
            jax.ShapeDtypeStruct((b, 1, m), jnp.float32),
            jax.ShapeDtypeStruct((b, 1, m), jnp.int32),
        ],
    )(xyz1, x2t)
    return d1[:, :, 0], d2[:, 0, :], i1[:, :, 0], i2[:, 0, :]


def _perm(v, idx):
    dnums = lax.GatherDimensionNumbers(
        offset_dims=(), collapsed_slice_dims=(0,), start_index_map=(0,))
    return lax.gather(v, jnp.reshape(idx, (_L, 1)), dnums, slice_sizes=(1,),
                      mode=lax.GatherScatterMode.PROMISE_IN_BOUNDS)


def _bcast(v, lane):
    return _perm(v, lax.iota(jnp.int32, _L) * 0 + lane)


def _lanemin(v):
    lanes = lax.iota(jnp.int32, _L)
    r = v
    for stride in (8, 4, 2, 1):
        r = jnp.minimum(r, _perm(r, lanes ^ stride))
    return r


def _rbf16(x):
    u = lax.bitcast_convert_type(x, jnp.uint32)
    r = u + jnp.uint32(0x7FFF) + ((u >> jnp.uint32(16)) & jnp.uint32(1))
    return lax.bitcast_convert_type(r & jnp.uint32(0xFFFF0000), jnp.float32)


def _prep(cx_ref, cy_ref, cz_ref, rx_ref, ry_ref, rz_ref, bb_ref, n):
    def body(t, carry):
        sl = pl.ds(t * _L, _L)
        cx = cx_ref[sl]
        cy = cy_ref[sl]
        cz = cz_ref[sl]
        rx_ref[sl] = _rbf16(cx)
        ry_ref[sl] = _rbf16(cy)
        rz_ref[sl] = _rbf16(cz)
        bb_ref[sl] = (cx * cx + cy * cy) + cz * cz
        return carry
    lax.fori_loop(0, n // _L, body, 0)


def _scan_fused(qx_ref, qy_ref, qz_ref, qrx_ref, qry_ref, qrz_ref, q0,
                crx_ref, cry_ref, crz_ref, cbb_ref,
                od_ref, oi_ref, colv_ref, coli_ref, nq, nc):
    nch = nc // _L
    lanes = lax.iota(jnp.int32, _L)
    big = jnp.full((_L,), jnp.float32(3.0e38))
    izero = jnp.zeros((_L,), jnp.int32)

    def block16(b, carry):
        qsl = pl.ds(q0 + b * _L, _L)
        qxv = qx_ref[qsl]
        qyv = qy_ref[qsl]
        qzv = qz_ref[qsl]
        aav16 = (qxv * qxv + qyv * qyv) + qzv * qzv
        qrxv = qrx_ref[qsl]
        qryv = qry_ref[qsl]
        qrzv = qrz_ref[qsl]
        dacc = jnp.zeros((_L,), jnp.float32)
        iacc = izero
        for sub in range(_L // _QG):
            bx = [2.0 * _bcast(qrxv, sub * _QG + k) for k in range(_QG)]
            by = [2.0 * _bcast(qryv, sub * _QG + k) for k in range(_QG)]
            bz = [2.0 * _bcast(qrzv, sub * _QG + k) for k in range(_QG)]
            av = [_bcast(aav16, sub * _QG + k) for k in range(_QG)]
            rid = [q0 + b * _L + (sub * _QG + k) for k in range(_QG)]

            def chunk(t, c, bx=bx, by=by, bz=bz, av=av, rid=rid):
                sl = pl.ds(t * _L, _L)
                cx = crx_ref[sl]
                cy = cry_ref[sl]
                cz = crz_ref[sl]
                bb = cbb_ref[sl]
                cv = colv_ref[sl]
                ci = coli_ref[sl]
                tv = jnp.full((_L,), t, dtype=jnp.int32)
                rms = list(c[:_QG])
                ris = list(c[_QG:])
                for k in range(_QG):
                    ab2 = (bx[k] * cx + by[k] * cy) + bz[k] * cz
                    d = jnp.maximum((av[k] + bb) - ab2, 0.0)
                    rbet = d < rms[k]
                    rms[k] = jnp.where(rbet, d, rms[k])
                    ris[k] = jnp.where(rbet, tv, ris[k])
                    cbet = d < cv
                    cv = jnp.where(cbet, d, cv)
                    ci = jnp.where(cbet,
                                   jnp.full((_L,), rid[k], dtype=jnp.int32),
                                   ci)
                colv_ref[sl] = cv
                coli_ref[sl] = ci
                return tuple(rms + ris)

            fin = lax.fori_loop(0, nch, chunk,
                                tuple([big] * _QG + [izero] * _QG))
            for k in range(_QG):
                rm = fin[k]
                ri = fin[_QG + k]
                pos = sub * _QG + k
                mvalv = _lanemin(rm)
                gidx = ri * _L + lanes
                cand = jnp.where(rm == mvalv, gidx,
                                 jnp.full((_L,), jnp.int32(1 << 30)))
                bidxv = _lanemin(cand)
                sel = lanes == pos
                dacc = jnp.where(sel, mvalv, dacc)
                iacc = jnp.where(sel, bidxv, iacc)
        od_ref[pl.ds(b * _L, _L)] = dacc
        oi_ref[pl.ds(b * _L, _L)] = iacc
        return carry

    lax.fori_loop(0, nq // _L, block16, 0)


def _worker_id():
    return lax.axis_index("c") * 16 + lax.axis_index("s")


def _sc_chamfer(x1x, x1y, x1z, x2x, x2y, x2z,
                d1_ref, d2_ref, i1_ref, i2_ref,
                c1x, c1y, c1z, c2x, c2y, c2z,
                r1x, r1y, r1z, r2x, r2y, r2z, b1, b2,
                colv, coli, od, oi, tv2, ti2, shv, shi):
    w = _worker_id()
    batch = w // _PARTS
    part = w % _PARTS
    cbase = batch * _N
    pltpu.sync_copy(x1x.at[pl.ds(cbase, _N)], c1x)
    pltpu.sync_copy(x1y.at[pl.ds(cbase, _N)], c1y)
    pltpu.sync_copy(x1z.at[pl.ds(cbase, _N)], c1z)
    pltpu.sync_copy(x2x.at[pl.ds(cbase, _N)], c2x)
    pltpu.sync_copy(x2y.at[pl.ds(cbase, _N)], c2y)
    pltpu.sync_copy(x2z.at[pl.ds(cbase, _N)], c2z)
    _prep(c1x, c1y, c1z, r1x, r1y, r1z, b1, _N)
    _prep(c2x, c2y, c2z, r2x, r2y, r2z, b2, _N)

    big = jnp.full((_L,), jnp.float32(3.0e38))
    izero = jnp.zeros((_L,), jnp.int32)

    def initcol(t, carry):
        sl = pl.ds(t * _L, _L)
        colv[sl] = big
        coli[sl] = izero
        return carry
    lax.fori_loop(0, _N // _L, initcol, 0)

    q0 = part * _QPW
    obase = w * _QPW

    _scan_fused(c1x, c1y, c1z, r1x, r1y, r1z, q0,
                r2x, r2y, r2z, b2, od, oi, colv, coli, _QPW, _N)
    pltpu.sync_copy(od, d1_ref.at[pl.ds(obase, _QPW)])
    pltpu.sync_copy(oi, i1_ref.at[pl.ds(obase, _QPW)])

    pltpu.sync_copy(colv, shv.at[part])
    pltpu.sync_copy(coli, shi.at[part])
    plsc.subcore_barrier()

    csl = pl.ds(part * _QPW, _QPW)
    pltpu.sync_copy(shv.at[0, csl], od)
    pltpu.sync_copy(shi.at[0, csl], oi)

    def merge_partial(p, carry):
        pltpu.sync_copy(shv.at[p, csl], tv2)
        pltpu.sync_copy(shi.at[p, csl], ti2)

        def merge_chunk(t, c):
            sl = pl.ds(t * _L, _L)
            cur = od[sl]
            new = tv2[sl]
            take = new < cur
            od[sl] = jnp.where(take, new, cur)
            oi[sl] = jnp.where(take, ti2[sl], oi[sl])
            return c
        lax.fori_loop(0, _QPW // _L, merge_chunk, 0)
        return carry
    lax.fori_loop(1, _PARTS, merge_partial, 0)

    pltpu.sync_copy(od, d2_ref.at[pl.ds(obase, _QPW)])
    pltpu.sync_copy(oi, i2_ref.at[pl.ds(obase, _QPW)])


def _sc_chamfer_call(xyz1, xyz2):
    b, n, _ = xyz1.shape
    x1 = jnp.transpose(xyz1, (2, 0, 1)).reshape(3, b * n)
    x2 = jnp.transpose(xyz2, (2, 0, 1)).reshape(3, b * n)
    mesh = plsc.VectorSubcoreMesh(core_axis_name="c", subcore_axis_name="s",
                                  num_cores=2, num_subcores=16)
    f = pl.kernel(
        _sc_chamfer,
        out_type=[
            jax.ShapeDtypeStruct((b * n,), jnp.float32),
            jax.ShapeDtypeStruct((b * n,), jnp.float32),
            jax.ShapeDtypeStruct((b * n,), jnp.int32),
            jax.ShapeDtypeStruct((b * n,), jnp.int32),
        ],
        mesh=mesh,
        scratch_types=(
            [pltpu.VMEM((n,), jnp.float32) for _ in range(14)]
            + [pltpu.VMEM((n,), jnp.float32),
               pltpu.VMEM((n,), jnp.int32),
               pltpu.VMEM((_QPW,), jnp.float32),
               pltpu.VMEM((_QPW,), jnp.int32),
               pltpu.VMEM((_QPW,), jnp.float32),
               pltpu.VMEM((_QPW,), jnp.int32),
               pltpu.VMEM_SHARED((_PARTS, n), jnp.float32),
               pltpu.VMEM_SHARED((_PARTS, n), jnp.int32)]),
    )
    d1, d2, i1, i2 = f(x1[0], x1[1], x1[2], x2[0], x2[1], x2[2])
    return (d1.reshape(b, n), d2.reshape(b, n),
            i1.reshape(b, n), i2.reshape(b, n))


def kernel(xyz1, xyz2):
    td1, td2, ti1, ti2 = _tc_chamfer(xyz1[:_NB_TC], xyz2[:_NB_TC])
    sd1, sd2, si1, si2 = _sc_chamfer_call(xyz1[_NB_TC:], xyz2[_NB_TC:])
    return (jnp.concatenate([td1, sd1], axis=0),
            jnp.concatenate([td2, sd2], axis=0),
            jnp.concatenate([ti1, si1], axis=0),
            jnp.concatenate([ti2, si2], axis=0))


# speedup vs baseline: 1.0870x; 1.0031x over previous
import jax
import jax.numpy as jnp
from jax import lax
from jax.experimental import pallas as pl
from jax.experimental.pallas import tpu as pltpu
from jax.experimental.pallas import tpu_sc as plsc

_B, _N = 8, 4096
_NB_SC = 2
_NB_TC = _B - _NB_SC
_L = 16
_W = 32
_QPW = (_NB_SC * _N) // _W
_PARTS = _N // _QPW
_QG = 4
_UNROLL = 1
_NBLK = 1024


def _tc_body(x1_ref, x2t_ref, d1_ref, i1_ref, d2_ref, i2_ref):
    i = pl.program_id(1)
    x1 = x1_ref[0]
    x2t = x2t_ref[0]
    m = x2t.shape[1]

    ab = jax.lax.dot_general(
        x1, x2t, dimension_numbers=(((1,), (0,)), ((), ())),
        preferred_element_type=jnp.float32)
    aa = jnp.sum(x1 * x1, axis=1, keepdims=True)
    bb = jnp.sum(x2t * x2t, axis=0, keepdims=True)
    d = jnp.maximum(aa + bb - 2.0 * ab, 0.0)

    rmin = jnp.min(d, axis=1, keepdims=True)
    lane = jax.lax.broadcasted_iota(jnp.int32, d.shape, 1)
    ridx = jnp.min(jnp.where(d == rmin, lane, jnp.int32(m)),
                   axis=1, keepdims=True)
    d1_ref[0] = rmin
    i1_ref[0] = ridx

    cmin = jnp.min(d, axis=0, keepdims=True)
    row = jax.lax.broadcasted_iota(jnp.int32, d.shape, 0) + i * _NBLK
    cidx = jnp.min(jnp.where(d == cmin, row, jnp.int32(1 << 30)),
                   axis=0, keepdims=True)

    @pl.when(i == 0)
    def _():
        d2_ref[0] = cmin
        i2_ref[0] = cidx

    @pl.when(i != 0)
    def _():
        prev_d = d2_ref[0]
        prev_i = i2_ref[0]
        take = cmin < prev_d
        d2_ref[0] = jnp.where(take, cmin, prev_d)
        i2_ref[0] = jnp.where(take, cidx, prev_i)


def _tc_chamfer(xyz1, xyz2):
    b, n, _ = xyz1.shape
    m = xyz2.shape[1]
    x2t = jnp.transpose(xyz2, (0, 2, 1))
    d1, i1, d2, i2 = pl.pallas_call(
        _tc_body,
        grid=(b, n // _NBLK),
        in_specs=[
            pl.BlockSpec((1, _NBLK, 3), lambda bi, ti: (bi, ti, 0)),
            pl.BlockSpec((1, 3, m), lambda bi, ti: (bi, 0, 0)),
        ],
        out_specs=[
            pl.BlockSpec((1, _NBLK, 1), lambda bi, ti: (bi, ti, 0)),
            pl.BlockSpec((1, _NBLK, 1), lambda bi, ti: (bi, ti, 0)),
            pl.BlockSpec((1, 1, m), lambda bi, ti: (bi, 0, 0)),
            pl.BlockSpec((1, 1, m), lambda bi, ti: (bi, 0, 0)),
        ],
        out_shape=[
            jax.ShapeDtypeStruct((b, n, 1), jnp.float32),
            jax.ShapeDtypeStruct((b, n, 1), jnp.int32),
            jax.ShapeDtypeStruct((b, 1, m), jnp.float32),
            jax.ShapeDtypeStruct((b, 1, m), jnp.int32),
        ],
    )(xyz1, x2t)
    return d1[:, :, 0], d2[:, 0, :], i1[:, :, 0], i2[:, 0, :]


def _perm(v, idx):
    dnums = lax.GatherDimensionNumbers(
        offset_dims=(), collapsed_slice_dims=(0,), start_index_map=(0,))
    return lax.gather(v, jnp.reshape(idx, (_L, 1)), dnums, slice_sizes=(1,),
                      mode=lax.GatherScatterMode.PROMISE_IN_BOUNDS)


def _bcast(v, lane):
    return _perm(v, lax.iota(jnp.int32, _L) * 0 + lane)


def _lanemin(v):
    lanes = lax.iota(jnp.int32, _L)
    r = v
    for stride in (8, 4, 2, 1):
        r = jnp.minimum(r, _perm(r, lanes ^ stride))
    return r


def _rbf16(x):
    u = lax.bitcast_convert_type(x, jnp.uint32)
    r = u + jnp.uint32(0x7FFF) + ((u >> jnp.uint32(16)) & jnp.uint32(1))
    return lax.bitcast_convert_type(r & jnp.uint32(0xFFFF0000), jnp.float32)


def _prep(cx_ref, cy_ref, cz_ref, rx_ref, ry_ref, rz_ref, bb_ref, n):
    def body(t, carry):
        sl = pl.ds(t * _L, _L)
        cx = cx_ref[sl]
        cy = cy_ref[sl]
        cz = cz_ref[sl]
        rx_ref[sl] = _rbf16(cx)
        ry_ref[sl] = _rbf16(cy)
        rz_ref[sl] = _rbf16(cz)
        bb_ref[sl] = (cx * cx + cy * cy) + cz * cz
        return carry
    lax.fori_loop(0, n // _L, body, 0)


def _scan_fused(qx_ref, qy_ref, qz_ref, qrx_ref, qry_ref, qrz_ref, q0,
                crx_ref, cry_ref, crz_ref, cbb_ref,
                od_ref, oi_ref, colv_ref, coli_ref, nq, nc):
    nch = nc // _L
    lanes = lax.iota(jnp.int32, _L)
    big = jnp.full((_L,), jnp.float32(3.0e38))
    izero = jnp.zeros((_L,), jnp.int32)

    def block16(b, carry):
        qsl = pl.ds(q0 + b * _L, _L)
        qxv = qx_ref[qsl]
        qyv = qy_ref[qsl]
        qzv = qz_ref[qsl]
        aav16 = (qxv * qxv + qyv * qyv) + qzv * qzv
        qrxv = qrx_ref[qsl]
        qryv = qry_ref[qsl]
        qrzv = qrz_ref[qsl]
        dacc = jnp.zeros((_L,), jnp.float32)
        iacc = izero
        for sub in range(_L // _QG):
            bx = [2.0 * _bcast(qrxv, sub * _QG + k) for k in range(_QG)]
            by = [2.0 * _bcast(qryv, sub * _QG + k) for k in range(_QG)]
            bz = [2.0 * _bcast(qrzv, sub * _QG + k) for k in range(_QG)]
            av = [_bcast(aav16, sub * _QG + k) for k in range(_QG)]
            rid = [q0 + b * _L + (sub * _QG + k) for k in range(_QG)]

            def chunk(t, c, bx=bx, by=by, bz=bz, av=av, rid=rid):
                sl = pl.ds(t * _L, _L)
                cx = crx_ref[sl]
                cy = cry_ref[sl]
                cz = crz_ref[sl]
                bb = cbb_ref[sl]
                cv = colv_ref[sl]
                ci = coli_ref[sl]
                tv = jnp.full((_L,), t, dtype=jnp.int32)
                rms = list(c[:_QG])
                ris = list(c[_QG:])
                for k in range(_QG):
                    ab2 = (bx[k] * cx + by[k] * cy) + bz[k] * cz
                    d = jnp.maximum((av[k] + bb) - ab2, 0.0)
                    rbet = d < rms[k]
                    rms[k] = jnp.where(rbet, d, rms[k])
                    ris[k] = jnp.where(rbet, tv, ris[k])
                    cbet = d < cv
                    cv = jnp.where(cbet, d, cv)
                    ci = jnp.where(cbet,
                                   jnp.full((_L,), rid[k], dtype=jnp.int32),
                                   ci)
                colv_ref[sl] = cv
                coli_ref[sl] = ci
                return tuple(rms + ris)

            fin = lax.fori_loop(0, nch, chunk,
                                tuple([big] * _QG + [izero] * _QG))
            for k in range(_QG):
                rm = fin[k]
                ri = fin[_QG + k]
                pos = sub * _QG + k
                mvalv = _lanemin(rm)
                gidx = ri * _L + lanes
                cand = jnp.where(rm == mvalv, gidx,
                                 jnp.full((_L,), jnp.int32(1 << 30)))
                bidxv = _lanemin(cand)
                sel = lanes == pos
                dacc = jnp.where(sel, mvalv, dacc)
                iacc = jnp.where(sel, bidxv, iacc)
        od_ref[pl.ds(b * _L, _L)] = dacc
        oi_ref[pl.ds(b * _L, _L)] = iacc
        return carry

    lax.fori_loop(0, nq // _L, block16, 0)


def _worker_id():
    return lax.axis_index("c") * 16 + lax.axis_index("s")


def _sc_chamfer(x1x, x1y, x1z, x2x, x2y, x2z,
                d1_ref, d2_ref, i1_ref, i2_ref,
                c1x, c1y, c1z, c2x, c2y, c2z,
                r1x, r1y, r1z, r2x, r2y, r2z, b1, b2,
                colv, coli, od, oi, tv2, ti2, shv, shi):
    w = _worker_id()
    batch = w // _PARTS
    part = w % _PARTS
    cbase = batch * _N
    pltpu.sync_copy(x1x.at[pl.ds(cbase, _N)], c1x)
    pltpu.sync_copy(x1y.at[pl.ds(cbase, _N)], c1y)
    pltpu.sync_copy(x1z.at[pl.ds(cbase, _N)], c1z)
    pltpu.sync_copy(x2x.at[pl.ds(cbase, _N)], c2x)
    pltpu.sync_copy(x2y.at[pl.ds(cbase, _N)], c2y)
    pltpu.sync_copy(x2z.at[pl.ds(cbase, _N)], c2z)
    _prep(c1x, c1y, c1z, r1x, r1y, r1z, b1, _N)
    _prep(c2x, c2y, c2z, r2x, r2y, r2z, b2, _N)

    big = jnp.full((_L,), jnp.float32(3.0e38))
    izero = jnp.zeros((_L,), jnp.int32)

    def initcol(t, carry):
        sl = pl.ds(t * _L, _L)
        colv[sl] = big
        coli[sl] = izero
        return carry
    lax.fori_loop(0, _N // _L, initcol, 0)

    q0 = part * _QPW
    obase = w * _QPW

    _scan_fused(c1x, c1y, c1z, r1x, r1y, r1z, q0,
                r2x, r2y, r2z, b2, od, oi, colv, coli, _QPW, _N)
    pltpu.sync_copy(od, d1_ref.at[pl.ds(obase, _QPW)])
    pltpu.sync_copy(oi, i1_ref.at[pl.ds(obase, _QPW)])

    pltpu.sync_copy(colv, shv.at[part])
    pltpu.sync_copy(coli, shi.at[part])
    plsc.subcore_barrier()

    csl = pl.ds(part * _QPW, _QPW)
    pltpu.sync_copy(shv.at[0, csl], od)
    pltpu.sync_copy(shi.at[0, csl], oi)

    def merge_partial(p, carry):
        pltpu.sync_copy(shv.at[p, csl], tv2)
        pltpu.sync_copy(shi.at[p, csl], ti2)

        def merge_chunk(t, c):
            sl = pl.ds(t * _L, _L)
            cur = od[sl]
            new = tv2[sl]
            take = new < cur
            od[sl] = jnp.where(take, new, cur)
            oi[sl] = jnp.where(take, ti2[sl], oi[sl])
            return c
        lax.fori_loop(0, _QPW // _L, merge_chunk, 0)
        return carry
    lax.fori_loop(1, _PARTS, merge_partial, 0)

    pltpu.sync_copy(od, d2_ref.at[pl.ds(obase, _QPW)])
    pltpu.sync_copy(oi, i2_ref.at[pl.ds(obase, _QPW)])


def _sc_chamfer_call(xyz1, xyz2):
    b, n, _ = xyz1.shape
    x1 = jnp.transpose(xyz1, (2, 0, 1)).reshape(3, b * n)
    x2 = jnp.transpose(xyz2, (2, 0, 1)).reshape(3, b * n)
    mesh = plsc.VectorSubcoreMesh(core_axis_name="c", subcore_axis_name="s",
                                  num_cores=2, num_subcores=16)
    f = pl.kernel(
        _sc_chamfer,
        out_type=[
            jax.ShapeDtypeStruct((b * n,), jnp.float32),
            jax.ShapeDtypeStruct((b * n,), jnp.float32),
            jax.ShapeDtypeStruct((b * n,), jnp.int32),
            jax.ShapeDtypeStruct((b * n,), jnp.int32),
        ],
        mesh=mesh,
        scratch_types=(
            [pltpu.VMEM((n,), jnp.float32) for _ in range(14)]
            + [pltpu.VMEM((n,), jnp.float32),
               pltpu.VMEM((n,), jnp.int32),
               pltpu.VMEM((_QPW,), jnp.float32),
               pltpu.VMEM((_QPW,), jnp.int32),
               pltpu.VMEM((_QPW,), jnp.float32),
               pltpu.VMEM((_QPW,), jnp.int32),
               pltpu.VMEM_SHARED((_PARTS, n), jnp.float32),
               pltpu.VMEM_SHARED((_PARTS, n), jnp.int32)]),
    )
    d1, d2, i1, i2 = f(x1[0], x1[1], x1[2], x2[0], x2[1], x2[2])
    return (d1.reshape(b, n), d2.reshape(b, n),
            i1.reshape(b, n), i2.reshape(b, n))


def kernel(xyz1, xyz2):
    td1, td2, ti1, ti2 = _tc_chamfer(xyz1[:_NB_TC], xyz2[:_NB_TC])
    sd1, sd2, si1, si2 = _sc_chamfer_call(xyz1[_NB_TC:], xyz2[_NB_TC:])
    return (jnp.concatenate([td1, sd1], axis=0),
            jnp.concatenate([td2, sd2], axis=0),
            jnp.concatenate([ti1, si1], axis=0),
            jnp.concatenate([ti2, si2], axis=0))


# speedup vs baseline: 1.0875x; 1.0005x over previous
import jax
import jax.numpy as jnp
from jax import lax
from jax.experimental import pallas as pl
from jax.experimental.pallas import tpu as pltpu
from jax.experimental.pallas import tpu_sc as plsc

_B, _N = 8, 4096
_NB_SC = 2
_NB_TC = _B - _NB_SC
_L = 16
_W = 32
_QPW = (_NB_SC * _N) // _W
_PARTS = _N // _QPW
_QG = 4
_UNROLL = 1
_NBLK = 1024


def _tc_body(x1_ref, x2t_ref, d1_ref, i1_ref, d2_ref, i2_ref):
    i = pl.program_id(1)
    x1 = x1_ref[0]
    x2t = x2t_ref[0]
    m = x2t.shape[1]

    ab = jax.lax.dot_general(
        x1, x2t, dimension_numbers=(((1,), (0,)), ((), ())),
        preferred_element_type=jnp.float32)
    aa = jnp.sum(x1 * x1, axis=1, keepdims=True)
    bb = jnp.sum(x2t * x2t, axis=0, keepdims=True)
    d = jnp.maximum(aa + bb - 2.0 * ab, 0.0)

    rmin = jnp.min(d, axis=1, keepdims=True)
    lane = jax.lax.broadcasted_iota(jnp.int32, d.shape, 1)
    ridx = jnp.min(jnp.where(d == rmin, lane, jnp.int32(m)),
                   axis=1, keepdims=True)
    d1_ref[0] = rmin
    i1_ref[0] = ridx

    cmin = jnp.min(d, axis=0, keepdims=True)
    row = jax.lax.broadcasted_iota(jnp.int32, d.shape, 0) + i * _NBLK
    cidx = jnp.min(jnp.where(d == cmin, row, jnp.int32(1 << 30)),
                   axis=0, keepdims=True)

    @pl.when(i == 0)
    def _():
        d2_ref[0] = cmin
        i2_ref[0] = cidx

    @pl.when(i != 0)
    def _():
        prev_d = d2_ref[0]
        prev_i = i2_ref[0]
        take = cmin < prev_d
        d2_ref[0] = jnp.where(take, cmin, prev_d)
        i2_ref[0] = jnp.where(take, cidx, prev_i)


def _tc_chamfer(xyz1, xyz2):
    b, n, _ = xyz1.shape
    m = xyz2.shape[1]
    x2t = jnp.transpose(xyz2, (0, 2, 1))
    d1, i1, d2, i2 = pl.pallas_call(
        _tc_body,
        grid=(b, n // _NBLK),
        in_specs=[
            pl.BlockSpec((1, _NBLK, 3), lambda bi, ti: (bi, ti, 0)),
            pl.BlockSpec((1, 3, m), lambda bi, ti: (bi, 0, 0)),
        ],
        out_specs=[
            pl.BlockSpec((1, _NBLK, 1), lambda bi, ti: (bi, ti, 0)),
            pl.BlockSpec((1, _NBLK, 1), lambda bi, ti: (bi, ti, 0)),
            pl.BlockSpec((1, 1, m), lambda bi, ti: (bi, 0, 0)),
            pl.BlockSpec((1, 1, m), lambda bi, ti: (bi, 0, 0)),
        ],
        out_shape=[
            jax.ShapeDtypeStruct((b, n, 1), jnp.float32),
            jax.ShapeDtypeStruct((b, n, 1), jnp.int32),
            jax.ShapeDtypeStruct((b, 1, m), jnp.float32),
            jax.ShapeDtypeStruct((b, 1, m), jnp.int32),
        ],
    )(xyz1, x2t)
    return d1[:, :, 0], d2[:, 0, :], i1[:, :, 0], i2[:, 0, :]


def _perm(v, idx):
    dnums = lax.GatherDimensionNumbers(
        offset_dims=(), collapsed_slice_dims=(0,), start_index_map=(0,))
    return lax.gather(v, jnp.reshape(idx, (_L, 1)), dnums, slice_sizes=(1,),
                      mode=lax.GatherScatterMode.PROMISE_IN_BOUNDS)


def _bcast(v, lane):
    return _perm(v, lax.iota(jnp.int32, _L) * 0 + lane)


def _lanemin(v):
    lanes = lax.iota(jnp.int32, _L)
    r = v
    for stride in (8, 4, 2, 1):
        r = jnp.minimum(r, _perm(r, lanes ^ stride))
    return r


def _rbf16(x):
    u = lax.bitcast_convert_type(x, jnp.uint32)
    r = u + jnp.uint32(0x7FFF) + ((u >> jnp.uint32(16)) & jnp.uint32(1))
    return lax.bitcast_convert_type(r & jnp.uint32(0xFFFF0000), jnp.float32)


def _prep(cx_ref, cy_ref, cz_ref, rx_ref, ry_ref, rz_ref, bb_ref, n):
    def body(t, carry):
        sl = pl.ds(t * _L, _L)
        cx = cx_ref[sl]
        cy = cy_ref[sl]
        cz = cz_ref[sl]
        rx_ref[sl] = _rbf16(cx)
        ry_ref[sl] = _rbf16(cy)
        rz_ref[sl] = _rbf16(cz)
        bb_ref[sl] = (cx * cx + cy * cy) + cz * cz
        return carry
    lax.fori_loop(0, n // _L, body, 0)


def _scan_fused(qx_ref, qy_ref, qz_ref, qrx_ref, qry_ref, qrz_ref, q0,
                crx_ref, cry_ref, crz_ref, cbb_ref,
                od_ref, oi_ref, colv_ref, coli_ref, nq, nc):
    nch = nc // _L
    lanes = lax.iota(jnp.int32, _L)
    big = jnp.full((_L,), jnp.float32(3.0e38))
    izero = jnp.zeros((_L,), jnp.int32)

    def block16(b, carry):
        qsl = pl.ds(q0 + b * _L, _L)
        qxv = qx_ref[qsl]
        qyv = qy_ref[qsl]
        qzv = qz_ref[qsl]
        aav16 = (qxv * qxv + qyv * qyv) + qzv * qzv
        qrxv = qrx_ref[qsl]
        qryv = qry_ref[qsl]
        qrzv = qrz_ref[qsl]
        dacc = jnp.zeros((_L,), jnp.float32)
        iacc = izero
        for sub in range(_L // _QG):
            bx = [2.0 * _bcast(qrxv, sub * _QG + k) for k in range(_QG)]
            by = [2.0 * _bcast(qryv, sub * _QG + k) for k in range(_QG)]
            bz = [2.0 * _bcast(qrzv, sub * _QG + k) for k in range(_QG)]
            av = [_bcast(aav16, sub * _QG + k) for k in range(_QG)]
            rid = [q0 + b * _L + (sub * _QG + k) for k in range(_QG)]

            def chunk(t, c, bx=bx, by=by, bz=bz, av=av, rid=rid):
                sl = pl.ds(t * _L, _L)
                cx = crx_ref[sl]
                cy = cry_ref[sl]
                cz = crz_ref[sl]
                bb = cbb_ref[sl]
                cv = colv_ref[sl]
                ci = coli_ref[sl]
                tv = jnp.full((_L,), t, dtype=jnp.int32)
                rms = list(c[:_QG])
                ris = list(c[_QG:])
                for k in range(_QG):
                    ab2 = (bx[k] * cx + by[k] * cy) + bz[k] * cz
                    d = jnp.maximum((av[k] + bb) - ab2, 0.0)
                    rbet = d < rms[k]
                    rms[k] = jnp.where(rbet, d, rms[k])
                    ris[k] = jnp.where(rbet, tv, ris[k])
                    cbet = d < cv
                    cv = jnp.where(cbet, d, cv)
                    ci = jnp.where(cbet,
                                   jnp.full((_L,), rid[k], dtype=jnp.int32),
                                   ci)
                colv_ref[sl] = cv
                coli_ref[sl] = ci
                return tuple(rms + ris)

            fin = plsc.parallel_loop(
                0, nch, 1, unroll=2,
                carry=tuple([big] * _QG + [izero] * _QG))(chunk)
            for k in range(_QG):
                rm = fin[k]
                ri = fin[_QG + k]
                pos = sub * _QG + k
                mvalv = _lanemin(rm)
                gidx = ri * _L + lanes
                cand = jnp.where(rm == mvalv, gidx,
                                 jnp.full((_L,), jnp.int32(1 << 30)))
                bidxv = _lanemin(cand)
                sel = lanes == pos
                dacc = jnp.where(sel, mvalv, dacc)
                iacc = jnp.where(sel, bidxv, iacc)
        od_ref[pl.ds(b * _L, _L)] = dacc
        oi_ref[pl.ds(b * _L, _L)] = iacc
        return carry

    lax.fori_loop(0, nq // _L, block16, 0)


def _worker_id():
    return lax.axis_index("c") * 16 + lax.axis_index("s")


def _sc_chamfer(x1x, x1y, x1z, x2x, x2y, x2z,
                d1_ref, d2_ref, i1_ref, i2_ref,
                c1x, c1y, c1z, c2x, c2y, c2z,
                r1x, r1y, r1z, r2x, r2y, r2z, b1, b2,
                colv, coli, od, oi, tv2, ti2, shv, shi):
    w = _worker_id()
    batch = w // _PARTS
    part = w % _PARTS
    cbase = batch * _N
    pltpu.sync_copy(x1x.at[pl.ds(cbase, _N)], c1x)
    pltpu.sync_copy(x1y.at[pl.ds(cbase, _N)], c1y)
    pltpu.sync_copy(x1z.at[pl.ds(cbase, _N)], c1z)
    pltpu.sync_copy(x2x.at[pl.ds(cbase, _N)], c2x)
    pltpu.sync_copy(x2y.at[pl.ds(cbase, _N)], c2y)
    pltpu.sync_copy(x2z.at[pl.ds(cbase, _N)], c2z)
    _prep(c1x, c1y, c1z, r1x, r1y, r1z, b1, _N)
    _prep(c2x, c2y, c2z, r2x, r2y, r2z, b2, _N)

    big = jnp.full((_L,), jnp.float32(3.0e38))
    izero = jnp.zeros((_L,), jnp.int32)

    def initcol(t, carry):
        sl = pl.ds(t * _L, _L)
        colv[sl] = big
        coli[sl] = izero
        return carry
    lax.fori_loop(0, _N // _L, initcol, 0)

    q0 = part * _QPW
    obase = w * _QPW

    _scan_fused(c1x, c1y, c1z, r1x, r1y, r1z, q0,
                r2x, r2y, r2z, b2, od, oi, colv, coli, _QPW, _N)
    pltpu.sync_copy(od, d1_ref.at[pl.ds(obase, _QPW)])
    pltpu.sync_copy(oi, i1_ref.at[pl.ds(obase, _QPW)])

    pltpu.sync_copy(colv, shv.at[part])
    pltpu.sync_copy(coli, shi.at[part])
    plsc.subcore_barrier()

    csl = pl.ds(part * _QPW, _QPW)
    pltpu.sync_copy(shv.at[0, csl], od)
    pltpu.sync_copy(shi.at[0, csl], oi)

    def merge_partial(p, carry):
        pltpu.sync_copy(shv.at[p, csl], tv2)
        pltpu.sync_copy(shi.at[p, csl], ti2)

        def merge_chunk(t, c):
            sl = pl.ds(t * _L, _L)
            cur = od[sl]
            new = tv2[sl]
            take = new < cur
            od[sl] = jnp.where(take, new, cur)
            oi[sl] = jnp.where(take, ti2[sl], oi[sl])
            return c
        lax.fori_loop(0, _QPW // _L, merge_chunk, 0)
        return carry
    lax.fori_loop(1, _PARTS, merge_partial, 0)

    pltpu.sync_copy(od, d2_ref.at[pl.ds(obase, _QPW)])
    pltpu.sync_copy(oi, i2_ref.at[pl.ds(obase, _QPW)])


def _sc_chamfer_call(xyz1, xyz2):
    b, n, _ = xyz1.shape
    x1 = jnp.transpose(xyz1, (2, 0, 1)).reshape(3, b * n)
    x2 = jnp.transpose(xyz2, (2, 0, 1)).reshape(3, b * n)
    mesh = plsc.VectorSubcoreMesh(core_axis_name="c", subcore_axis_name="s",
                                  num_cores=2, num_subcores=16)
    f = pl.kernel(
        _sc_chamfer,
        out_type=[
            jax.ShapeDtypeStruct((b * n,), jnp.float32),
            jax.ShapeDtypeStruct((b * n,), jnp.float32),
            jax.ShapeDtypeStruct((b * n,), jnp.int32),
            jax.ShapeDtypeStruct((b * n,), jnp.int32),
        ],
        mesh=mesh,
        scratch_types=(
            [pltpu.VMEM((n,), jnp.float32) for _ in range(14)]
            + [pltpu.VMEM((n,), jnp.float32),
               pltpu.VMEM((n,), jnp.int32),
               pltpu.VMEM((_QPW,), jnp.float32),
               pltpu.VMEM((_QPW,), jnp.int32),
               pltpu.VMEM((_QPW,), jnp.float32),
               pltpu.VMEM((_QPW,), jnp.int32),
               pltpu.VMEM_SHARED((_PARTS, n), jnp.float32),
               pltpu.VMEM_SHARED((_PARTS, n), jnp.int32)]),
    )
    d1, d2, i1, i2 = f(x1[0], x1[1], x1[2], x2[0], x2[1], x2[2])
    return (d1.reshape(b, n), d2.reshape(b, n),
            i1.reshape(b, n), i2.reshape(b, n))


def kernel(xyz1, xyz2):
    td1, td2, ti1, ti2 = _tc_chamfer(xyz1[:_NB_TC], xyz2[:_NB_TC])
    sd1, sd2, si1, si2 = _sc_chamfer_call(xyz1[_NB_TC:], xyz2[_NB_TC:])
    return (jnp.concatenate([td1, sd1], axis=0),
            jnp.concatenate([td2, sd2], axis=0),
            jnp.concatenate([ti1, si1], axis=0),
            jnp.concatenate([ti2, si2], axis=0))


# speedup vs baseline: 1.0876x; 1.0001x over previous
import jax
import jax.numpy as jnp
from jax import lax
from jax.experimental import pallas as pl
from jax.experimental.pallas import tpu as pltpu
from jax.experimental.pallas import tpu_sc as plsc

_B, _N = 8, 4096
_NB_SC = 2
_NB_TC = _B - _NB_SC
_L = 16
_W = 32
_QPW = (_NB_SC * _N) // _W
_PARTS = _N // _QPW
_QG = 4
_UNROLL = 1
_NBLK = 1024


def _tc_body(x1_ref, x2t_ref, d1_ref, i1_ref, d2_ref, i2_ref):
    i = pl.program_id(1)
    x1 = x1_ref[0]
    x2t = x2t_ref[0]
    m = x2t.shape[1]

    ab = jax.lax.dot_general(
        x1, x2t, dimension_numbers=(((1,), (0,)), ((), ())),
        preferred_element_type=jnp.float32)
    aa = jnp.sum(x1 * x1, axis=1, keepdims=True)
    bb = jnp.sum(x2t * x2t, axis=0, keepdims=True)
    d = jnp.maximum(aa + bb - 2.0 * ab, 0.0)

    rmin = jnp.min(d, axis=1, keepdims=True)
    lane = jax.lax.broadcasted_iota(jnp.int32, d.shape, 1)
    ridx = jnp.min(jnp.where(d == rmin, lane, jnp.int32(m)),
                   axis=1, keepdims=True)
    d1_ref[0] = rmin
    i1_ref[0] = ridx

    cmin = jnp.min(d, axis=0, keepdims=True)
    row = jax.lax.broadcasted_iota(jnp.int32, d.shape, 0) + i * _NBLK
    cidx = jnp.min(jnp.where(d == cmin, row, jnp.int32(1 << 30)),
                   axis=0, keepdims=True)

    @pl.when(i == 0)
    def _():
        d2_ref[0] = cmin
        i2_ref[0] = cidx

    @pl.when(i != 0)
    def _():
        prev_d = d2_ref[0]
        prev_i = i2_ref[0]
        take = cmin < prev_d
        d2_ref[0] = jnp.where(take, cmin, prev_d)
        i2_ref[0] = jnp.where(take, cidx, prev_i)


def _tc_chamfer(xyz1, xyz2):
    b, n, _ = xyz1.shape
    m = xyz2.shape[1]
    x2t = jnp.transpose(xyz2, (0, 2, 1))
    d1, i1, d2, i2 = pl.pallas_call(
        _tc_body,
        grid=(b, n // _NBLK),
        in_specs=[
            pl.BlockSpec((1, _NBLK, 3), lambda bi, ti: (bi, ti, 0)),
            pl.BlockSpec((1, 3, m), lambda bi, ti: (bi, 0, 0)),
        ],
        out_specs=[
            pl.BlockSpec((1, _NBLK, 1), lambda bi, ti: (bi, ti, 0)),
            pl.BlockSpec((1, _NBLK, 1), lambda bi, ti: (bi, ti, 0)),
            pl.BlockSpec((1, 1, m), lambda bi, ti: (bi, 0, 0)),
            pl.BlockSpec((1, 1, m), lambda bi, ti: (bi, 0, 0)),
        ],
        out_shape=[
            jax.ShapeDtypeStruct((b, n, 1), jnp.float32),
            jax.ShapeDtypeStruct((b, n, 1), jnp.int32),
            jax.ShapeDtypeStruct((b, 1, m), jnp.float32),
            jax.ShapeDtypeStruct((b, 1, m), jnp.int32),
        ],
    )(xyz1, x2t)
    return d1[:, :, 0], d2[:, 0, :], i1[:, :, 0], i2[:, 0, :]


def _perm(v, idx):
    dnums = lax.GatherDimensionNumbers(
        offset_dims=(), collapsed_slice_dims=(0,), start_index_map=(0,))
    return lax.gather(v, jnp.reshape(idx, (_L, 1)), dnums, slice_sizes=(1,),
                      mode=lax.GatherScatterMode.PROMISE_IN_BOUNDS)


def _bcast(v, lane):
    return _perm(v, lax.iota(jnp.int32, _L) * 0 + lane)


def _lanemin(v):
    lanes = lax.iota(jnp.int32, _L)
    r = v
    for stride in (8, 4, 2, 1):
        r = jnp.minimum(r, _perm(r, lanes ^ stride))
    return r


def _rbf16(x):
    u = lax.bitcast_convert_type(x, jnp.uint32)
    r = u + jnp.uint32(0x7FFF) + ((u >> jnp.uint32(16)) & jnp.uint32(1))
    return lax.bitcast_convert_type(r & jnp.uint32(0xFFFF0000), jnp.float32)


def _prep(cx_ref, cy_ref, cz_ref, rx_ref, ry_ref, rz_ref, bb_ref, n):
    def body(t, carry):
        sl = pl.ds(t * _L, _L)
        cx = cx_ref[sl]
        cy = cy_ref[sl]
        cz = cz_ref[sl]
        rx_ref[sl] = _rbf16(cx)
        ry_ref[sl] = _rbf16(cy)
        rz_ref[sl] = _rbf16(cz)
        bb_ref[sl] = (cx * cx + cy * cy) + cz * cz
        return carry
    lax.fori_loop(0, n // _L, body, 0)


def _scan_fused(qx_ref, qy_ref, qz_ref, qrx_ref, qry_ref, qrz_ref, q0,
                crx_ref, cry_ref, crz_ref, cbb_ref,
                od_ref, oi_ref, colv_ref, coli_ref, nq, nc):
    nch = nc // _L
    lanes = lax.iota(jnp.int32, _L)
    big = jnp.full((_L,), jnp.float32(3.0e38))
    izero = jnp.zeros((_L,), jnp.int32)

    def block16(b, carry):
        qsl = pl.ds(q0 + b * _L, _L)
        qxv = qx_ref[qsl]
        qyv = qy_ref[qsl]
        qzv = qz_ref[qsl]
        aav16 = (qxv * qxv + qyv * qyv) + qzv * qzv
        qrxv = qrx_ref[qsl]
        qryv = qry_ref[qsl]
        qrzv = qrz_ref[qsl]
        dacc = jnp.zeros((_L,), jnp.float32)
        iacc = izero
        for sub in range(_L // _QG):
            bx = [2.0 * _bcast(qrxv, sub * _QG + k) for k in range(_QG)]
            by = [2.0 * _bcast(qryv, sub * _QG + k) for k in range(_QG)]
            bz = [2.0 * _bcast(qrzv, sub * _QG + k) for k in range(_QG)]
            av = [_bcast(aav16, sub * _QG + k) for k in range(_QG)]
            rid = [q0 + b * _L + (sub * _QG + k) for k in range(_QG)]

            def chunk(t, c, bx=bx, by=by, bz=bz, av=av, rid=rid):
                sl = pl.ds(t * _L, _L)
                cx = crx_ref[sl]
                cy = cry_ref[sl]
                cz = crz_ref[sl]
                bb = cbb_ref[sl]
                cv = colv_ref[sl]
                ci = coli_ref[sl]
                tv = jnp.full((_L,), t, dtype=jnp.int32)
                rms = list(c[:_QG])
                ris = list(c[_QG:])
                for k in range(_QG):
                    ab2 = (bx[k] * cx + by[k] * cy) + bz[k] * cz
                    d = jnp.maximum((av[k] + bb) - ab2, 0.0)
                    rbet = d < rms[k]
                    rms[k] = jnp.where(rbet, d, rms[k])
                    ris[k] = jnp.where(rbet, tv, ris[k])
                    cbet = d < cv
                    cv = jnp.where(cbet, d, cv)
                    ci = jnp.where(cbet,
                                   jnp.full((_L,), rid[k], dtype=jnp.int32),
                                   ci)
                colv_ref[sl] = cv
                coli_ref[sl] = ci
                return tuple(rms + ris)

            fin = lax.fori_loop(0, nch, chunk,
                                tuple([big] * _QG + [izero] * _QG))
            for k in range(_QG):
                rm = fin[k]
                ri = fin[_QG + k]
                pos = sub * _QG + k
                mvalv = _lanemin(rm)
                gidx = ri * _L + lanes
                cand = jnp.where(rm == mvalv, gidx,
                                 jnp.full((_L,), jnp.int32(1 << 30)))
                bidxv = _lanemin(cand)
                sel = lanes == pos
                dacc = jnp.where(sel, mvalv, dacc)
                iacc = jnp.where(sel, bidxv, iacc)
        od_ref[pl.ds(b * _L, _L)] = dacc
        oi_ref[pl.ds(b * _L, _L)] = iacc
        return carry

    lax.fori_loop(0, nq // _L, block16, 0)


def _worker_id():
    return lax.axis_index("c") * 16 + lax.axis_index("s")


def _sc_chamfer(x1x, x1y, x1z, x2x, x2y, x2z,
                d1_ref, d2_ref, i1_ref, i2_ref,
                c1x, c1y, c1z, c2x, c2y, c2z,
                r1x, r1y, r1z, r2x, r2y, r2z, b1, b2,
                colv, coli, od, oi, tv2, ti2, shv, shi):
    w = _worker_id()
    batch = w // _PARTS
    part = w % _PARTS
    cbase = batch * _N
    pltpu.sync_copy(x1x.at[pl.ds(cbase, _N)], c1x)
    pltpu.sync_copy(x1y.at[pl.ds(cbase, _N)], c1y)
    pltpu.sync_copy(x1z.at[pl.ds(cbase, _N)], c1z)
    pltpu.sync_copy(x2x.at[pl.ds(cbase, _N)], c2x)
    pltpu.sync_copy(x2y.at[pl.ds(cbase, _N)], c2y)
    pltpu.sync_copy(x2z.at[pl.ds(cbase, _N)], c2z)
    _prep(c1x, c1y, c1z, r1x, r1y, r1z, b1, _N)
    _prep(c2x, c2y, c2z, r2x, r2y, r2z, b2, _N)

    big = jnp.full((_L,), jnp.float32(3.0e38))
    izero = jnp.zeros((_L,), jnp.int32)

    def initcol(t, carry):
        sl = pl.ds(t * _L, _L)
        colv[sl] = big
        coli[sl] = izero
        return carry
    lax.fori_loop(0, _N // _L, initcol, 0)

    q0 = part * _QPW
    obase = w * _QPW

    _scan_fused(c1x, c1y, c1z, r1x, r1y, r1z, q0,
                r2x, r2y, r2z, b2, od, oi, colv, coli, _QPW, _N)
    pltpu.sync_copy(od, d1_ref.at[pl.ds(obase, _QPW)])
    pltpu.sync_copy(oi, i1_ref.at[pl.ds(obase, _QPW)])

    pltpu.sync_copy(colv, shv.at[part])
    pltpu.sync_copy(coli, shi.at[part])
    plsc.subcore_barrier()

    csl = pl.ds(part * _QPW, _QPW)
    pltpu.sync_copy(shv.at[0, csl], od)
    pltpu.sync_copy(shi.at[0, csl], oi)

    def merge_partial(p, carry):
        pltpu.sync_copy(shv.at[p, csl], tv2)
        pltpu.sync_copy(shi.at[p, csl], ti2)

        def merge_chunk(t, c):
            sl = pl.ds(t * _L, _L)
            cur = od[sl]
            new = tv2[sl]
            take = new < cur
            od[sl] = jnp.where(take, new, cur)
            oi[sl] = jnp.where(take, ti2[sl], oi[sl])
            return c
        lax.fori_loop(0, _QPW // _L, merge_chunk, 0)
        return carry
    lax.fori_loop(1, _PARTS, merge_partial, 0)

    pltpu.sync_copy(od, d2_ref.at[pl.ds(obase, _QPW)])
    pltpu.sync_copy(oi, i2_ref.at[pl.ds(obase, _QPW)])


def _sc_chamfer_call(xyz1, xyz2):
    b, n, _ = xyz1.shape
    x1 = jnp.transpose(xyz1, (2, 0, 1)).reshape(3, b * n)
    x2 = jnp.transpose(xyz2, (2, 0, 1)).reshape(3, b * n)
    mesh = plsc.VectorSubcoreMesh(core_axis_name="c", subcore_axis_name="s",
                                  num_cores=2, num_subcores=16)
    f = pl.kernel(
        _sc_chamfer,
        out_type=[
            jax.ShapeDtypeStruct((b * n,), jnp.float32),
            jax.ShapeDtypeStruct((b * n,), jnp.float32),
            jax.ShapeDtypeStruct((b * n,), jnp.int32),
            jax.ShapeDtypeStruct((b * n,), jnp.int32),
        ],
        mesh=mesh,
        scratch_types=(
            [pltpu.VMEM((n,), jnp.float32) for _ in range(14)]
            + [pltpu.VMEM((n,), jnp.float32),
               pltpu.VMEM((n,), jnp.int32),
               pltpu.VMEM((_QPW,), jnp.float32),
               pltpu.VMEM((_QPW,), jnp.int32),
               pltpu.VMEM((_QPW,), jnp.float32),
               pltpu.VMEM((_QPW,), jnp.int32),
               pltpu.VMEM_SHARED((_PARTS, n), jnp.float32),
               pltpu.VMEM_SHARED((_PARTS, n), jnp.int32)]),
    )
    d1, d2, i1, i2 = f(x1[0], x1[1], x1[2], x2[0], x2[1], x2[2])
    return (d1.reshape(b, n), d2.reshape(b, n),
            i1.reshape(b, n), i2.reshape(b, n))


def kernel(xyz1, xyz2):
    td1, td2, ti1, ti2 = _tc_chamfer(xyz1[:_NB_TC], xyz2[:_NB_TC])
    sd1, sd2, si1, si2 = _sc_chamfer_call(xyz1[_NB_TC:], xyz2[_NB_TC:])
    return (jnp.concatenate([td1, sd1], axis=0),
            jnp.concatenate([td2, sd2], axis=0),
            jnp.concatenate([ti1, si1], axis=0),
            jnp.concatenate([ti2, si2], axis=0))
